# baseline (speedup 1.0000x reference)
import functools

import jax
import jax.numpy as jnp
from jax import lax
from jax.experimental import pallas as pl
from jax.experimental.pallas import tpu as pltpu
from jax.experimental.pallas import tpu_sc as plsc

N = 10000
E = 320000
H = 128

NC = 2
NS = 16
NW = NC * NS

RPW = 320
SCROWS = NS * RPW
ACC_N = 5248
ZR = ACC_N // NS

INC = 4096
NIC = (E + INC - 1) // INC
E_IN = NIC * INC
PAD_DST = 16383

FB = 4096
SB = FB + 256 + 16
SEL_CAP = (NIC + 1) * INC

K = 128

_sc_mesh = plsc.VectorSubcoreMesh(core_axis_name="c", subcore_axis_name="s")


@functools.partial(
    pl.kernel,
    out_type=[
        jax.ShapeDtypeStruct((NW * SEL_CAP,), jnp.int32),
        jax.ShapeDtypeStruct((NW * 16,), jnp.int32),
    ],
    mesh=_sc_mesh,
    compiler_params=pltpu.CompilerParams(needs_layout_passes=False),
    scratch_types=[
        pltpu.VMEM((2 * INC,), jnp.int32),
        pltpu.VMEM((SB,), jnp.int32),
        pltpu.VMEM((16,), jnp.int32),
        pltpu.SemaphoreType.DMA,
    ],
)
def _select(edges_in, tr_pack, sel_pack, counts,
            in_p, sb_p, cnt_v, sem_a):
    cid = lax.axis_index("c")
    sid = lax.axis_index("s")
    wid = cid * NS + sid
    off = cid * SCROWS
    glo = wid * RPW
    ghi = glo + RPW

    selbase = wid * SEL_CAP

    pltpu.sync_copy(tr_pack.at[pl.ds(wid * SB, SB)], sb_p)

    pltpu.async_copy(edges_in.at[pl.ds(0, INC)], in_p.at[pl.ds(0, INC)], sem_a)

    def chunk_body(c, carry):
        bc0, ct0 = carry
        cb = lax.rem(c, 2)
        pltpu.make_async_copy(edges_in.at[pl.ds(c * INC, INC)], in_p.at[pl.ds(cb * INC, INC)], sem_a).wait()

        @pl.when(c + 1 < NIC)
        def _():
            nb = lax.rem(c + 1, 2)
            pltpu.async_copy(edges_in.at[pl.ds((c + 1) * INC, INC)], in_p.at[pl.ds(nb * INC, INC)], sem_a)

        def grp_body(g, carry2):
            bc, ct = carry2
            base = cb * INC + g * 256
            for v in range(16):
                o = base + v * 16
                p = in_p[pl.ds(o, 16)]
                pd = p & 16383
                m = (pd >= glo) & (pd < ghi)
                mi = jnp.where(m, 1, 0)
                cs = jnp.cumsum(mi)
                pos = (bc - 1) + cs
                packed = ((p >> 14) << 13) | (pd - off)
                plsc.store_scatter(sb_p, [pos], packed, mask=m)
                bc = bc + jnp.max(plsc.all_reduce_population_count(m))

            need = bc >= FB

            @pl.when(need)
            def _():
                cta = pl.multiple_of(selbase + ct, FB)
                pltpu.sync_copy(sb_p.at[pl.ds(0, FB)], sel_pack.at[pl.ds(cta, FB)])
                for t in range(17):
                    sb_p[pl.ds(16 * t, 16)] = sb_p[pl.ds(FB + 16 * t, 16)]
                pltpu.sync_copy(tr_pack.at[pl.ds(wid * SB + 272, SB - 272)],
                                sb_p.at[pl.ds(272, SB - 272)])

            bc = jnp.where(need, bc - FB, bc)
            ct = jnp.where(need, ct + FB, ct)
            return (bc, ct)

        return lax.fori_loop(0, 16, grp_body, (bc0, ct0))

    bc, ct = lax.fori_loop(0, NIC, chunk_body,
                           (jnp.int32(0), jnp.int32(0)))

    cta = pl.multiple_of(selbase + ct, FB)
    pltpu.sync_copy(sb_p.at[pl.ds(0, FB)], sel_pack.at[pl.ds(cta, FB)])

    total = ct + bc
    cnt_v[...] = total * jnp.ones((16,), jnp.int32)
    pltpu.sync_copy(cnt_v, counts.at[pl.ds(wid * 16, 16)])


@functools.partial(
    pl.kernel,
    out_type=jax.ShapeDtypeStruct((NC, SCROWS, H), jnp.float32),
    mesh=_sc_mesh,
    compiler_params=pltpu.CompilerParams(needs_layout_passes=False),
    scratch_types=[
        pltpu.VMEM((K,), jnp.int32),
        pltpu.VMEM((K,), jnp.int32),
        pltpu.VMEM((1, K), jnp.int32),
        pltpu.VMEM((1, K), jnp.int32),
        pltpu.VMEM((1, K), jnp.int32),
        pltpu.VMEM((1, K), jnp.int32),
        pltpu.VMEM((K, H), jnp.float32),
        pltpu.VMEM((K, H), jnp.float32),
        pltpu.VMEM((16,), jnp.int32),
        pltpu.VMEM_SHARED((ACC_N, H), jnp.float32),
        pltpu.SemaphoreType.DMA,
        pltpu.SemaphoreType.DMA,
    ],
)
def _segsum(h_hbm, sel_pack, counts, zero_hbm, out_hbm,
            pk0, pk1, si0, si1, di0, di1, rows0, rows1, cnt_v, acc, sem0, sem1):
    cid = lax.axis_index("c")
    sid = lax.axis_index("s")
    wid = cid * NS + sid

    pltpu.sync_copy(zero_hbm, acc.at[pl.ds(sid * ZR, ZR)])
    plsc.subcore_barrier()

    selbase = wid * SEL_CAP
    pltpu.sync_copy(counts.at[pl.ds(wid * 16, 16)], cnt_v)
    n_chunks = lax.div(jnp.max(cnt_v[...]) + (K - 1), K)
    npairs = lax.div(n_chunks + 1, 2)

    pk = (pk0, pk1)
    si = (si0, si1)
    di = (di0, di1)
    rows = (rows0, rows1)
    sems = (sem0, sem1)

    def load_unpack(o, b):
        pltpu.sync_copy(sel_pack.at[pl.ds(o, K)], pk[b])
        for t in range(K // 16):
            p = pk[b][pl.ds(16 * t, 16)]
            si[b][0, pl.ds(16 * t, 16)] = p >> 13
            di[b][0, pl.ds(16 * t, 16)] = p & 8191

    @pl.when(n_chunks > 0)
    def _():
        load_unpack(pl.multiple_of(selbase, K), 0)
        pltpu.async_copy(h_hbm.at[si0.at[0]], rows0, sem0)

    def step(j, b):
        @pl.when(j < n_chunks)
        def _():
            pltpu.make_async_copy(h_hbm.at[si[b].at[0]], rows[b], sems[b]).wait()

            @pl.when(j + 1 < n_chunks)
            def _():
                nb = 1 - b
                load_unpack(pl.multiple_of(selbase + (j + 1) * K, K), nb)
                pltpu.async_copy(h_hbm.at[si[nb].at[0]], rows[nb], sems[nb])

            pltpu.sync_copy(rows[b], acc.at[di[b].at[0]], add=True)

    def pair_body(p, carry):
        step(2 * p, 0)
        step(2 * p + 1, 1)
        return carry

    lax.fori_loop(0, npairs, pair_body, jnp.int32(0))
    plsc.subcore_barrier()

    pltpu.sync_copy(
        acc.at[pl.ds(sid * RPW, RPW)],
        out_hbm.at[cid, pl.ds(sid * RPW, RPW)],
    )


R = 2000
NB = N // R


def _dense_a_body(h_ref, a_ref, w1_ref, b1_ref, w2_ref, b2_ref, u_ref, st_ref):
    i = pl.program_id(0)
    hs = h_ref[...] + a_ref[...]
    t = jnp.maximum(
        jnp.dot(hs, w1_ref[...], preferred_element_type=jnp.float32) + b1_ref[...], 0.0
    )
    u = jnp.dot(t, w2_ref[...], preferred_element_type=jnp.float32) + b2_ref[...]
    u_ref[...] = u
    st = jnp.sum(u, 0, keepdims=True)

    @pl.when(i == 0)
    def _():
        st_ref[...] = st

    @pl.when(i != 0)
    def _():
        st_ref[...] = st_ref[...] + st


_dense_a = pl.pallas_call(
    _dense_a_body,
    grid=(NB,),
    in_specs=[
        pl.BlockSpec((R, H), lambda i: (i, 0)),
        pl.BlockSpec((R, H), lambda i: (i, 0)),
        pl.BlockSpec((H, H), lambda i: (0, 0)),
        pl.BlockSpec((1, H), lambda i: (0, 0)),
        pl.BlockSpec((H, H), lambda i: (0, 0)),
        pl.BlockSpec((1, H), lambda i: (0, 0)),
    ],
    out_specs=[
        pl.BlockSpec((R, H), lambda i: (i, 0)),
        pl.BlockSpec((1, H), lambda i: (0, 0)),
    ],
    out_shape=[
        jax.ShapeDtypeStruct((N, H), jnp.float32),
        jax.ShapeDtypeStruct((1, H), jnp.float32),
    ],
)


def _dense_v_body(u_ref, st_ref, v_ref):
    i = pl.program_id(0)
    d = u_ref[...] - st_ref[...] / N
    v = jnp.sum(d * d, 0, keepdims=True)

    @pl.when(i == 0)
    def _():
        v_ref[...] = v

    @pl.when(i != 0)
    def _():
        v_ref[...] = v_ref[...] + v


_dense_v = pl.pallas_call(
    _dense_v_body,
    grid=(NB,),
    in_specs=[
        pl.BlockSpec((R, H), lambda i: (i, 0)),
        pl.BlockSpec((1, H), lambda i: (0, 0)),
    ],
    out_specs=pl.BlockSpec((1, H), lambda i: (0, 0)),
    out_shape=jax.ShapeDtypeStruct((1, H), jnp.float32),
)


def _dense_b_body(u_ref, st_ref, v_ref, g_ref, be_ref, h_ref):
    mean = st_ref[...] / N
    var = v_ref[...] / N
    h_ref[...] = jnp.maximum(
        (u_ref[...] - mean) / jnp.sqrt(var + 1e-5) * g_ref[...] + be_ref[...], 0.0
    )


_dense_b = pl.pallas_call(
    _dense_b_body,
    grid=(NB,),
    in_specs=[
        pl.BlockSpec((R, H), lambda i: (i, 0)),
        pl.BlockSpec((1, H), lambda i: (0, 0)),
        pl.BlockSpec((1, H), lambda i: (0, 0)),
        pl.BlockSpec((1, H), lambda i: (0, 0)),
        pl.BlockSpec((1, H), lambda i: (0, 0)),
    ],
    out_specs=pl.BlockSpec((R, H), lambda i: (i, 0)),
    out_shape=jax.ShapeDtypeStruct((N, H), jnp.float32),
)


def _dense_bf_body(u_ref, st_ref, v_ref, g_ref, be_ref, wf_ref, bf_ref, o_ref):
    mean = st_ref[...] / N
    var = v_ref[...] / N
    hn = jnp.maximum((u_ref[...] - mean) / jnp.sqrt(var + 1e-5) * g_ref[...] + be_ref[...], 0.0)
    o_ref[...] = (
        jnp.dot(hn, wf_ref[...], preferred_element_type=jnp.float32) + bf_ref[...]
    )


_dense_bf = pl.pallas_call(
    _dense_bf_body,
    grid=(NB,),
    in_specs=[
        pl.BlockSpec((R, H), lambda i: (i, 0)),
        pl.BlockSpec((1, H), lambda i: (0, 0)),
        pl.BlockSpec((1, H), lambda i: (0, 0)),
        pl.BlockSpec((1, H), lambda i: (0, 0)),
        pl.BlockSpec((1, H), lambda i: (0, 0)),
        pl.BlockSpec((H, 1), lambda i: (0, 0)),
        pl.BlockSpec((1, 1), lambda i: (0, 0)),
    ],
    out_specs=pl.BlockSpec((R, 1), lambda i: (i, 0)),
    out_shape=jax.ShapeDtypeStruct((N, 1), jnp.float32),
)


def kernel(x, edge_index, w1_0, b1_0, w2_0, b2_0, g_0, be_0, w1_1, b1_1, w2_1, b2_1,
           g_1, be_1, w1_2, b1_2, w2_2, b2_2, g_2, be_2, wf, bf):
    src = edge_index[0].astype(jnp.int32)
    dst = edge_index[1].astype(jnp.int32)
    pad_in = E_IN - E
    packed_edges = (src << 14) | dst
    edges_in = jnp.concatenate([packed_edges, jnp.full((pad_in,), PAD_DST, jnp.int32)])

    widv = jnp.arange(NW, dtype=jnp.int32)
    tr_s = (widv * 311) % N
    tr_d = SCROWS + (widv % NS)
    tr_pack = jnp.broadcast_to(((tr_s << 13) | tr_d)[:, None], (NW, SB)).reshape(-1)
    zeros = jnp.zeros((ZR, H), jnp.float32)

    sel_pack, counts = _select(edges_in, tr_pack)

    params = [
        (w1_0, b1_0, w2_0, b2_0, g_0, be_0),
        (w1_1, b1_1, w2_1, b2_1, g_1, be_1),
        (w1_2, b1_2, w2_2, b2_2, g_2, be_2),
    ]

    h = x
    out = None
    for l, (w1, b1, w2, b2, g, be) in enumerate(params):
        p = _segsum(h, sel_pack, counts, zeros)
        agg = jnp.concatenate([p[0], p[1][: N - SCROWS]], axis=0)
        u, st = _dense_a(h, agg, w1, b1.reshape(1, H), w2, b2.reshape(1, H))
        v = _dense_v(u, st)
        if l < 2:
            h = _dense_b(u, st, v, g.reshape(1, H), be.reshape(1, H))
        else:
            out = _dense_bf(u, st, v, g.reshape(1, H), be.reshape(1, H),
                            wf, bf.reshape(1, 1))
    return out


# speedup vs baseline: 1.0306x; 1.0306x over previous
import functools

import jax
import jax.numpy as jnp
from jax import lax
from jax.experimental import pallas as pl
from jax.experimental.pallas import tpu as pltpu
from jax.experimental.pallas import tpu_sc as plsc

N = 10000
E = 320000
H = 128

NC = 2
NS = 16
NW = NC * NS

RPW = 320
SCROWS = NS * RPW
ACC_N = 5248
ZR = ACC_N // NS

INC = 4096
NIC = (E + INC - 1) // INC
E_IN = NIC * INC
PAD_DST = 16383

FB = 4096
SB = FB + 256 + 16
SEL_CAP = (NIC + 1) * INC

K = 128

_sc_mesh = plsc.VectorSubcoreMesh(core_axis_name="c", subcore_axis_name="s")


@functools.partial(
    pl.kernel,
    out_type=[
        jax.ShapeDtypeStruct((NW * SEL_CAP,), jnp.int32),
        jax.ShapeDtypeStruct((NW * 16,), jnp.int32),
    ],
    mesh=_sc_mesh,
    compiler_params=pltpu.CompilerParams(needs_layout_passes=False),
    scratch_types=[
        pltpu.VMEM((2 * INC,), jnp.int32),
        pltpu.VMEM((SB,), jnp.int32),
        pltpu.VMEM((16,), jnp.int32),
        pltpu.SemaphoreType.DMA,
    ],
)
def _select(edges_in, tr_pack, sel_pack, counts,
            in_p, sb_p, cnt_v, sem_a):
    cid = lax.axis_index("c")
    sid = lax.axis_index("s")
    wid = cid * NS + sid
    off = cid * SCROWS
    glo = wid * RPW
    ghi = glo + RPW

    selbase = wid * SEL_CAP

    pltpu.sync_copy(tr_pack.at[pl.ds(wid * SB, SB)], sb_p)

    pltpu.async_copy(edges_in.at[pl.ds(0, INC)], in_p.at[pl.ds(0, INC)], sem_a)

    def chunk_body(c, carry):
        bc0, ct0 = carry
        cb = lax.rem(c, 2)
        pltpu.make_async_copy(edges_in.at[pl.ds(c * INC, INC)], in_p.at[pl.ds(cb * INC, INC)], sem_a).wait()

        @pl.when(c + 1 < NIC)
        def _():
            nb = lax.rem(c + 1, 2)
            pltpu.async_copy(edges_in.at[pl.ds((c + 1) * INC, INC)], in_p.at[pl.ds(nb * INC, INC)], sem_a)

        def grp_body(g, carry2):
            bc, ct = carry2
            base = cb * INC + g * 256
            for v in range(16):
                o = base + v * 16
                p = in_p[pl.ds(o, 16)]
                pd = p & 16383
                m = (pd >= glo) & (pd < ghi)
                mi = jnp.where(m, 1, 0)
                cs = jnp.cumsum(mi)
                pos = (bc - 1) + cs
                packed = ((p >> 14) << 13) | (pd - off)
                plsc.store_scatter(sb_p, [pos], packed, mask=m)
                bc = bc + cs[15]

            need = bc >= FB

            @pl.when(need)
            def _():
                cta = pl.multiple_of(selbase + ct, FB)
                pltpu.sync_copy(sb_p.at[pl.ds(0, FB)], sel_pack.at[pl.ds(cta, FB)])
                for t in range(17):
                    sb_p[pl.ds(16 * t, 16)] = sb_p[pl.ds(FB + 16 * t, 16)]
                pltpu.sync_copy(tr_pack.at[pl.ds(wid * SB + 272, SB - 272)],
                                sb_p.at[pl.ds(272, SB - 272)])

            bc = jnp.where(need, bc - FB, bc)
            ct = jnp.where(need, ct + FB, ct)
            return (bc, ct)

        return lax.fori_loop(0, 16, grp_body, (bc0, ct0))

    bc, ct = lax.fori_loop(0, NIC, chunk_body,
                           (jnp.int32(0), jnp.int32(0)))

    cta = pl.multiple_of(selbase + ct, FB)
    pltpu.sync_copy(sb_p.at[pl.ds(0, FB)], sel_pack.at[pl.ds(cta, FB)])

    total = ct + bc
    cnt_v[...] = total * jnp.ones((16,), jnp.int32)
    pltpu.sync_copy(cnt_v, counts.at[pl.ds(wid * 16, 16)])


@functools.partial(
    pl.kernel,
    out_type=jax.ShapeDtypeStruct((NC, SCROWS, H), jnp.float32),
    mesh=_sc_mesh,
    compiler_params=pltpu.CompilerParams(needs_layout_passes=False),
    scratch_types=[
        pltpu.VMEM((K,), jnp.int32),
        pltpu.VMEM((K,), jnp.int32),
        pltpu.VMEM((1, K), jnp.int32),
        pltpu.VMEM((1, K), jnp.int32),
        pltpu.VMEM((1, K), jnp.int32),
        pltpu.VMEM((1, K), jnp.int32),
        pltpu.VMEM((K, H), jnp.float32),
        pltpu.VMEM((K, H), jnp.float32),
        pltpu.VMEM((16,), jnp.int32),
        pltpu.VMEM_SHARED((ACC_N, H), jnp.float32),
        pltpu.SemaphoreType.DMA,
        pltpu.SemaphoreType.DMA,
    ],
)
def _segsum(h_hbm, sel_pack, counts, zero_hbm, out_hbm,
            pk0, pk1, si0, si1, di0, di1, rows0, rows1, cnt_v, acc, sem0, sem1):
    cid = lax.axis_index("c")
    sid = lax.axis_index("s")
    wid = cid * NS + sid

    pltpu.sync_copy(zero_hbm, acc.at[pl.ds(sid * ZR, ZR)])
    plsc.subcore_barrier()

    selbase = wid * SEL_CAP
    pltpu.sync_copy(counts.at[pl.ds(wid * 16, 16)], cnt_v)
    n_chunks = lax.div(jnp.max(cnt_v[...]) + (K - 1), K)
    npairs = lax.div(n_chunks + 1, 2)

    pk = (pk0, pk1)
    si = (si0, si1)
    di = (di0, di1)
    rows = (rows0, rows1)
    sems = (sem0, sem1)

    def load_unpack(o, b):
        pltpu.sync_copy(sel_pack.at[pl.ds(o, K)], pk[b])
        for t in range(K // 16):
            p = pk[b][pl.ds(16 * t, 16)]
            si[b][0, pl.ds(16 * t, 16)] = p >> 13
            di[b][0, pl.ds(16 * t, 16)] = p & 8191

    @pl.when(n_chunks > 0)
    def _():
        load_unpack(pl.multiple_of(selbase, K), 0)
        pltpu.async_copy(h_hbm.at[si0.at[0]], rows0, sem0)

    def step(j, b):
        @pl.when(j < n_chunks)
        def _():
            pltpu.make_async_copy(h_hbm.at[si[b].at[0]], rows[b], sems[b]).wait()

            @pl.when(j + 1 < n_chunks)
            def _():
                nb = 1 - b
                load_unpack(pl.multiple_of(selbase + (j + 1) * K, K), nb)
                pltpu.async_copy(h_hbm.at[si[nb].at[0]], rows[nb], sems[nb])

            pltpu.sync_copy(rows[b], acc.at[di[b].at[0]], add=True)

    def pair_body(p, carry):
        step(2 * p, 0)
        step(2 * p + 1, 1)
        return carry

    lax.fori_loop(0, npairs, pair_body, jnp.int32(0))
    plsc.subcore_barrier()

    pltpu.sync_copy(
        acc.at[pl.ds(sid * RPW, RPW)],
        out_hbm.at[cid, pl.ds(sid * RPW, RPW)],
    )


R = 2000
NB = N // R


def _dense_a_body(h_ref, a_ref, w1_ref, b1_ref, w2_ref, b2_ref, u_ref, st_ref):
    i = pl.program_id(0)
    hs = h_ref[...] + a_ref[...]
    t = jnp.maximum(
        jnp.dot(hs, w1_ref[...], preferred_element_type=jnp.float32) + b1_ref[...], 0.0
    )
    u = jnp.dot(t, w2_ref[...], preferred_element_type=jnp.float32) + b2_ref[...]
    u_ref[...] = u
    st = jnp.sum(u, 0, keepdims=True)

    @pl.when(i == 0)
    def _():
        st_ref[...] = st

    @pl.when(i != 0)
    def _():
        st_ref[...] = st_ref[...] + st


_dense_a = pl.pallas_call(
    _dense_a_body,
    grid=(NB,),
    in_specs=[
        pl.BlockSpec((R, H), lambda i: (i, 0)),
        pl.BlockSpec((R, H), lambda i: (i, 0)),
        pl.BlockSpec((H, H), lambda i: (0, 0)),
        pl.BlockSpec((1, H), lambda i: (0, 0)),
        pl.BlockSpec((H, H), lambda i: (0, 0)),
        pl.BlockSpec((1, H), lambda i: (0, 0)),
    ],
    out_specs=[
        pl.BlockSpec((R, H), lambda i: (i, 0)),
        pl.BlockSpec((1, H), lambda i: (0, 0)),
    ],
    out_shape=[
        jax.ShapeDtypeStruct((N, H), jnp.float32),
        jax.ShapeDtypeStruct((1, H), jnp.float32),
    ],
)


def _dense_v_body(u_ref, st_ref, v_ref):
    i = pl.program_id(0)
    d = u_ref[...] - st_ref[...] / N
    v = jnp.sum(d * d, 0, keepdims=True)

    @pl.when(i == 0)
    def _():
        v_ref[...] = v

    @pl.when(i != 0)
    def _():
        v_ref[...] = v_ref[...] + v


_dense_v = pl.pallas_call(
    _dense_v_body,
    grid=(NB,),
    in_specs=[
        pl.BlockSpec((R, H), lambda i: (i, 0)),
        pl.BlockSpec((1, H), lambda i: (0, 0)),
    ],
    out_specs=pl.BlockSpec((1, H), lambda i: (0, 0)),
    out_shape=jax.ShapeDtypeStruct((1, H), jnp.float32),
)


def _dense_b_body(u_ref, st_ref, v_ref, g_ref, be_ref, h_ref):
    mean = st_ref[...] / N
    var = v_ref[...] / N
    h_ref[...] = jnp.maximum(
        (u_ref[...] - mean) / jnp.sqrt(var + 1e-5) * g_ref[...] + be_ref[...], 0.0
    )


_dense_b = pl.pallas_call(
    _dense_b_body,
    grid=(NB,),
    in_specs=[
        pl.BlockSpec((R, H), lambda i: (i, 0)),
        pl.BlockSpec((1, H), lambda i: (0, 0)),
        pl.BlockSpec((1, H), lambda i: (0, 0)),
        pl.BlockSpec((1, H), lambda i: (0, 0)),
        pl.BlockSpec((1, H), lambda i: (0, 0)),
    ],
    out_specs=pl.BlockSpec((R, H), lambda i: (i, 0)),
    out_shape=jax.ShapeDtypeStruct((N, H), jnp.float32),
)


def _dense_bf_body(u_ref, st_ref, v_ref, g_ref, be_ref, wf_ref, bf_ref, o_ref):
    mean = st_ref[...] / N
    var = v_ref[...] / N
    hn = jnp.maximum((u_ref[...] - mean) / jnp.sqrt(var + 1e-5) * g_ref[...] + be_ref[...], 0.0)
    o_ref[...] = (
        jnp.dot(hn, wf_ref[...], preferred_element_type=jnp.float32) + bf_ref[...]
    )


_dense_bf = pl.pallas_call(
    _dense_bf_body,
    grid=(NB,),
    in_specs=[
        pl.BlockSpec((R, H), lambda i: (i, 0)),
        pl.BlockSpec((1, H), lambda i: (0, 0)),
        pl.BlockSpec((1, H), lambda i: (0, 0)),
        pl.BlockSpec((1, H), lambda i: (0, 0)),
        pl.BlockSpec((1, H), lambda i: (0, 0)),
        pl.BlockSpec((H, 1), lambda i: (0, 0)),
        pl.BlockSpec((1, 1), lambda i: (0, 0)),
    ],
    out_specs=pl.BlockSpec((R, 1), lambda i: (i, 0)),
    out_shape=jax.ShapeDtypeStruct((N, 1), jnp.float32),
)


def kernel(x, edge_index, w1_0, b1_0, w2_0, b2_0, g_0, be_0, w1_1, b1_1, w2_1, b2_1,
           g_1, be_1, w1_2, b1_2, w2_2, b2_2, g_2, be_2, wf, bf):
    src = edge_index[0].astype(jnp.int32)
    dst = edge_index[1].astype(jnp.int32)
    pad_in = E_IN - E
    packed_edges = (src << 14) | dst
    edges_in = jnp.concatenate([packed_edges, jnp.full((pad_in,), PAD_DST, jnp.int32)])

    widv = jnp.arange(NW, dtype=jnp.int32)
    tr_s = (widv * 311) % N
    tr_d = SCROWS + (widv % NS)
    tr_pack = jnp.broadcast_to(((tr_s << 13) | tr_d)[:, None], (NW, SB)).reshape(-1)
    zeros = jnp.zeros((ZR, H), jnp.float32)

    sel_pack, counts = _select(edges_in, tr_pack)

    params = [
        (w1_0, b1_0, w2_0, b2_0, g_0, be_0),
        (w1_1, b1_1, w2_1, b2_1, g_1, be_1),
        (w1_2, b1_2, w2_2, b2_2, g_2, be_2),
    ]

    h = x
    out = None
    for l, (w1, b1, w2, b2, g, be) in enumerate(params):
        p = _segsum(h, sel_pack, counts, zeros)
        agg = jnp.concatenate([p[0], p[1][: N - SCROWS]], axis=0)
        u, st = _dense_a(h, agg, w1, b1.reshape(1, H), w2, b2.reshape(1, H))
        v = _dense_v(u, st)
        if l < 2:
            h = _dense_b(u, st, v, g.reshape(1, H), be.reshape(1, H))
        else:
            out = _dense_bf(u, st, v, g.reshape(1, H), be.reshape(1, H),
                            wf, bf.reshape(1, 1))
    return out


# speedup vs baseline: 1.0333x; 1.0026x over previous
import functools

import jax
import jax.numpy as jnp
from jax import lax
from jax.experimental import pallas as pl
from jax.experimental.pallas import tpu as pltpu
from jax.experimental.pallas import tpu_sc as plsc

N = 10000
E = 320000
H = 128

NC = 2
NS = 16
NW = NC * NS

RPW = 320
SCROWS = NS * RPW
ACC_N = 5248
ZR = ACC_N // NS

INC = 4096
NIC = (E + INC - 1) // INC
E_IN = NIC * INC
PAD_DST = 16383

FB = 4096
SB = FB + 256 + 16
SEL_CAP = (NIC + 1) * INC

K = 128

_sc_mesh = plsc.VectorSubcoreMesh(core_axis_name="c", subcore_axis_name="s")


@functools.partial(
    pl.kernel,
    out_type=[
        jax.ShapeDtypeStruct((NW * SEL_CAP,), jnp.int32),
        jax.ShapeDtypeStruct((NW * 16,), jnp.int32),
    ],
    mesh=_sc_mesh,
    compiler_params=pltpu.CompilerParams(needs_layout_passes=False),
    scratch_types=[
        pltpu.VMEM((2 * INC,), jnp.int32),
        pltpu.VMEM((SB,), jnp.int32),
        pltpu.VMEM((16,), jnp.int32),
        pltpu.SemaphoreType.DMA,
    ],
)
def _select(edges_in, tr_pack, sel_pack, counts,
            in_p, sb_p, cnt_v, sem_a):
    cid = lax.axis_index("c")
    sid = lax.axis_index("s")
    wid = cid * NS + sid
    off = cid * SCROWS
    glo = wid * RPW
    ghi = glo + RPW

    selbase = wid * SEL_CAP

    pltpu.sync_copy(tr_pack.at[pl.ds(wid * SB, SB)], sb_p)

    pltpu.async_copy(edges_in.at[pl.ds(0, INC)], in_p.at[pl.ds(0, INC)], sem_a)

    def chunk_body(c, carry):
        bc0, ct0 = carry
        cb = lax.rem(c, 2)
        pltpu.make_async_copy(edges_in.at[pl.ds(c * INC, INC)], in_p.at[pl.ds(cb * INC, INC)], sem_a).wait()

        @pl.when(c + 1 < NIC)
        def _():
            nb = lax.rem(c + 1, 2)
            pltpu.async_copy(edges_in.at[pl.ds((c + 1) * INC, INC)], in_p.at[pl.ds(nb * INC, INC)], sem_a)

        def grp_body(g, carry2):
            bc, ct = carry2
            base = cb * INC + g * 256
            for v in range(16):
                o = base + v * 16
                p = in_p[pl.ds(o, 16)]
                pd = p & 16383
                m = (pd >= glo) & (pd < ghi)
                mi = jnp.where(m, 1, 0)
                cs = jnp.cumsum(mi)
                pos = (bc - 1) + cs
                packed = ((p >> 14) << 13) | (pd - off)
                plsc.store_scatter(sb_p, [pos], packed, mask=m)
                bc = bc + cs[15]

            need = bc >= FB

            @pl.when(need)
            def _():
                cta = pl.multiple_of(selbase + ct, FB)
                pltpu.sync_copy(sb_p.at[pl.ds(0, FB)], sel_pack.at[pl.ds(cta, FB)])
                for t in range(17):
                    sb_p[pl.ds(16 * t, 16)] = sb_p[pl.ds(FB + 16 * t, 16)]
                pltpu.sync_copy(tr_pack.at[pl.ds(wid * SB + 272, SB - 272)],
                                sb_p.at[pl.ds(272, SB - 272)])

            bc = jnp.where(need, bc - FB, bc)
            ct = jnp.where(need, ct + FB, ct)
            return (bc, ct)

        return lax.fori_loop(0, 16, grp_body, (bc0, ct0))

    bc, ct = lax.fori_loop(0, NIC, chunk_body,
                           (jnp.int32(0), jnp.int32(0)))

    cta = pl.multiple_of(selbase + ct, FB)
    pltpu.sync_copy(sb_p.at[pl.ds(0, FB)], sel_pack.at[pl.ds(cta, FB)])

    total = ct + bc
    cnt_v[...] = total * jnp.ones((16,), jnp.int32)
    pltpu.sync_copy(cnt_v, counts.at[pl.ds(wid * 16, 16)])


@functools.partial(
    pl.kernel,
    out_type=jax.ShapeDtypeStruct((NC, SCROWS, H), jnp.float32),
    mesh=_sc_mesh,
    compiler_params=pltpu.CompilerParams(needs_layout_passes=False),
    scratch_types=[
        pltpu.VMEM((K,), jnp.int32),
        pltpu.VMEM((K,), jnp.int32),
        pltpu.VMEM((1, K), jnp.int32),
        pltpu.VMEM((1, K), jnp.int32),
        pltpu.VMEM((1, K), jnp.int32),
        pltpu.VMEM((1, K), jnp.int32),
        pltpu.VMEM((K, H), jnp.float32),
        pltpu.VMEM((K, H), jnp.float32),
        pltpu.VMEM((16,), jnp.int32),
        pltpu.VMEM_SHARED((ACC_N, H), jnp.float32),
        pltpu.SemaphoreType.DMA,
        pltpu.SemaphoreType.DMA,
        pltpu.SemaphoreType.DMA,
        pltpu.SemaphoreType.DMA,
    ],
)
def _segsum(h_hbm, sel_pack, counts, zero_hbm, out_hbm,
            pk0, pk1, si0, si1, di0, di1, rows0, rows1, cnt_v, acc, sem0, sem1,
            ssem0, ssem1):
    cid = lax.axis_index("c")
    sid = lax.axis_index("s")
    wid = cid * NS + sid

    pltpu.sync_copy(zero_hbm, acc.at[pl.ds(sid * ZR, ZR)])
    plsc.subcore_barrier()

    selbase = wid * SEL_CAP
    pltpu.sync_copy(counts.at[pl.ds(wid * 16, 16)], cnt_v)
    n_chunks = lax.div(jnp.max(cnt_v[...]) + (K - 1), K)
    npairs = lax.div(n_chunks + 1, 2)

    pk = (pk0, pk1)
    si = (si0, si1)
    di = (di0, di1)
    rows = (rows0, rows1)
    sems = (sem0, sem1)
    ssems = (ssem0, ssem1)

    def load_unpack(o, b):
        pltpu.sync_copy(sel_pack.at[pl.ds(o, K)], pk[b])
        for t in range(K // 16):
            p = pk[b][pl.ds(16 * t, 16)]
            si[b][0, pl.ds(16 * t, 16)] = p >> 13
            di[b][0, pl.ds(16 * t, 16)] = p & 8191

    @pl.when(n_chunks > 0)
    def _():
        load_unpack(pl.multiple_of(selbase, K), 0)
        pltpu.async_copy(h_hbm.at[si0.at[0]], rows0, sem0)

    def step(j, b):
        @pl.when(j < n_chunks)
        def _():
            nb = 1 - b
            pltpu.make_async_copy(h_hbm.at[si[b].at[0]], rows[b], sems[b]).wait()
            pltpu.async_copy(rows[b], acc.at[di[b].at[0]], ssems[b], add=True)

            @pl.when(j + 1 < n_chunks)
            def _():
                @pl.when(j >= 1)
                def _():
                    pltpu.make_async_copy(rows[nb], acc.at[di[nb].at[0]], ssems[nb]).wait()

                load_unpack(pl.multiple_of(selbase + (j + 1) * K, K), nb)
                pltpu.async_copy(h_hbm.at[si[nb].at[0]], rows[nb], sems[nb])

    def pair_body(p, carry):
        step(2 * p, 0)
        step(2 * p + 1, 1)
        return carry

    lax.fori_loop(0, npairs, pair_body, jnp.int32(0))

    @pl.when(n_chunks > 1)
    def _():
        b2 = lax.rem(n_chunks, 2)
        @pl.when(b2 == 0)
        def _():
            pltpu.make_async_copy(rows0, acc.at[di0.at[0]], ssem0).wait()

        @pl.when(b2 == 1)
        def _():
            pltpu.make_async_copy(rows1, acc.at[di1.at[0]], ssem1).wait()

    @pl.when(n_chunks > 0)
    def _():
        b1 = lax.rem(n_chunks - 1, 2)

        @pl.when(b1 == 0)
        def _():
            pltpu.make_async_copy(rows0, acc.at[di0.at[0]], ssem0).wait()

        @pl.when(b1 == 1)
        def _():
            pltpu.make_async_copy(rows1, acc.at[di1.at[0]], ssem1).wait()

    plsc.subcore_barrier()

    pltpu.sync_copy(
        acc.at[pl.ds(sid * RPW, RPW)],
        out_hbm.at[cid, pl.ds(sid * RPW, RPW)],
    )


R = 2000
NB = N // R


def _dense_a_body(h_ref, a_ref, w1_ref, b1_ref, w2_ref, b2_ref, u_ref, st_ref):
    i = pl.program_id(0)
    hs = h_ref[...] + a_ref[...]
    t = jnp.maximum(
        jnp.dot(hs, w1_ref[...], preferred_element_type=jnp.float32) + b1_ref[...], 0.0
    )
    u = jnp.dot(t, w2_ref[...], preferred_element_type=jnp.float32) + b2_ref[...]
    u_ref[...] = u
    st = jnp.sum(u, 0, keepdims=True)

    @pl.when(i == 0)
    def _():
        st_ref[...] = st

    @pl.when(i != 0)
    def _():
        st_ref[...] = st_ref[...] + st


_dense_a = pl.pallas_call(
    _dense_a_body,
    grid=(NB,),
    in_specs=[
        pl.BlockSpec((R, H), lambda i: (i, 0)),
        pl.BlockSpec((R, H), lambda i: (i, 0)),
        pl.BlockSpec((H, H), lambda i: (0, 0)),
        pl.BlockSpec((1, H), lambda i: (0, 0)),
        pl.BlockSpec((H, H), lambda i: (0, 0)),
        pl.BlockSpec((1, H), lambda i: (0, 0)),
    ],
    out_specs=[
        pl.BlockSpec((R, H), lambda i: (i, 0)),
        pl.BlockSpec((1, H), lambda i: (0, 0)),
    ],
    out_shape=[
        jax.ShapeDtypeStruct((N, H), jnp.float32),
        jax.ShapeDtypeStruct((1, H), jnp.float32),
    ],
)


def _dense_v_body(u_ref, st_ref, v_ref):
    i = pl.program_id(0)
    d = u_ref[...] - st_ref[...] / N
    v = jnp.sum(d * d, 0, keepdims=True)

    @pl.when(i == 0)
    def _():
        v_ref[...] = v

    @pl.when(i != 0)
    def _():
        v_ref[...] = v_ref[...] + v


_dense_v = pl.pallas_call(
    _dense_v_body,
    grid=(NB,),
    in_specs=[
        pl.BlockSpec((R, H), lambda i: (i, 0)),
        pl.BlockSpec((1, H), lambda i: (0, 0)),
    ],
    out_specs=pl.BlockSpec((1, H), lambda i: (0, 0)),
    out_shape=jax.ShapeDtypeStruct((1, H), jnp.float32),
)


def _dense_b_body(u_ref, st_ref, v_ref, g_ref, be_ref, h_ref):
    mean = st_ref[...] / N
    var = v_ref[...] / N
    h_ref[...] = jnp.maximum(
        (u_ref[...] - mean) / jnp.sqrt(var + 1e-5) * g_ref[...] + be_ref[...], 0.0
    )


_dense_b = pl.pallas_call(
    _dense_b_body,
    grid=(NB,),
    in_specs=[
        pl.BlockSpec((R, H), lambda i: (i, 0)),
        pl.BlockSpec((1, H), lambda i: (0, 0)),
        pl.BlockSpec((1, H), lambda i: (0, 0)),
        pl.BlockSpec((1, H), lambda i: (0, 0)),
        pl.BlockSpec((1, H), lambda i: (0, 0)),
    ],
    out_specs=pl.BlockSpec((R, H), lambda i: (i, 0)),
    out_shape=jax.ShapeDtypeStruct((N, H), jnp.float32),
)


def _dense_bf_body(u_ref, st_ref, v_ref, g_ref, be_ref, wf_ref, bf_ref, o_ref):
    mean = st_ref[...] / N
    var = v_ref[...] / N
    hn = jnp.maximum((u_ref[...] - mean) / jnp.sqrt(var + 1e-5) * g_ref[...] + be_ref[...], 0.0)
    o_ref[...] = (
        jnp.dot(hn, wf_ref[...], preferred_element_type=jnp.float32) + bf_ref[...]
    )


_dense_bf = pl.pallas_call(
    _dense_bf_body,
    grid=(NB,),
    in_specs=[
        pl.BlockSpec((R, H), lambda i: (i, 0)),
        pl.BlockSpec((1, H), lambda i: (0, 0)),
        pl.BlockSpec((1, H), lambda i: (0, 0)),
        pl.BlockSpec((1, H), lambda i: (0, 0)),
        pl.BlockSpec((1, H), lambda i: (0, 0)),
        pl.BlockSpec((H, 1), lambda i: (0, 0)),
        pl.BlockSpec((1, 1), lambda i: (0, 0)),
    ],
    out_specs=pl.BlockSpec((R, 1), lambda i: (i, 0)),
    out_shape=jax.ShapeDtypeStruct((N, 1), jnp.float32),
)


def kernel(x, edge_index, w1_0, b1_0, w2_0, b2_0, g_0, be_0, w1_1, b1_1, w2_1, b2_1,
           g_1, be_1, w1_2, b1_2, w2_2, b2_2, g_2, be_2, wf, bf):
    src = edge_index[0].astype(jnp.int32)
    dst = edge_index[1].astype(jnp.int32)
    pad_in = E_IN - E
    packed_edges = (src << 14) | dst
    edges_in = jnp.concatenate([packed_edges, jnp.full((pad_in,), PAD_DST, jnp.int32)])

    widv = jnp.arange(NW, dtype=jnp.int32)
    tr_s = (widv * 311) % N
    tr_d = SCROWS + (widv % NS)
    tr_pack = jnp.broadcast_to(((tr_s << 13) | tr_d)[:, None], (NW, SB)).reshape(-1)
    zeros = jnp.zeros((ZR, H), jnp.float32)

    sel_pack, counts = _select(edges_in, tr_pack)

    params = [
        (w1_0, b1_0, w2_0, b2_0, g_0, be_0),
        (w1_1, b1_1, w2_1, b2_1, g_1, be_1),
        (w1_2, b1_2, w2_2, b2_2, g_2, be_2),
    ]

    h = x
    out = None
    for l, (w1, b1, w2, b2, g, be) in enumerate(params):
        p = _segsum(h, sel_pack, counts, zeros)
        agg = jnp.concatenate([p[0], p[1][: N - SCROWS]], axis=0)
        u, st = _dense_a(h, agg, w1, b1.reshape(1, H), w2, b2.reshape(1, H))
        v = _dense_v(u, st)
        if l < 2:
            h = _dense_b(u, st, v, g.reshape(1, H), be.reshape(1, H))
        else:
            out = _dense_bf(u, st, v, g.reshape(1, H), be.reshape(1, H),
                            wf, bf.reshape(1, 1))
    return out
